# Initial kernel scaffold; baseline (speedup 1.0000x reference)
import jax
import jax.numpy as jnp
from jax.experimental import pallas as pl


def kernel(z, W):
    raise NotImplementedError("write your pallas kernel here")


# speedup vs baseline: 1.2181x; 1.2181x over previous
import jax
import jax.numpy as jnp
from jax.experimental import pallas as pl
from jax.experimental.pallas import tpu as pltpu

CB = 1024


def _vq_body(zsq_ref, z_ref, w_ref, q_ref, idx_ref, loss_ref):
    zb = z_ref[0]
    w = w_ref[...]
    t = zb.shape[1]
    s = jax.lax.dot_general(w.astype(jnp.bfloat16), zb.astype(jnp.bfloat16),
                            (((1,), (0,)), ((), ())),
                            preferred_element_type=jnp.float32)
    zsq = zsq_ref[0]
    wsq = jnp.sum(w * w, axis=1, keepdims=True)
    d2 = (zsq + wsq) - 2.0 * s
    dist = jnp.sqrt(jnp.maximum(d2, 0.0))
    m = jnp.min(dist, axis=0, keepdims=True)
    iota = jax.lax.broadcasted_iota(jnp.int32, (CB, t), 0)
    idx = jnp.min(jnp.where(dist == m, iota, CB), axis=0).astype(jnp.int32)
    idx_ref[0, 0, :] = idx
    onehot = (iota == idx[None, :]).astype(jnp.float32)
    q = jax.lax.dot_general(w, onehot, (((0,), (0,)), ((), ())),
                            preferred_element_type=jnp.float32,
                            precision=jax.lax.Precision.HIGHEST)
    q_ref[0] = q
    loss_ref[0, 0, :] = jnp.sum((zb - q) ** 2, axis=0)


@jax.jit
def kernel(z, W):
    B, d, T = z.shape
    z_flat = jnp.transpose(z, (0, 2, 1)).reshape(-1, d)
    zsq = jnp.sum(z_flat * z_flat, axis=1).reshape(B, 1, T)
    tb = 512
    q, idx3, lossp = pl.pallas_call(
        _vq_body,
        grid=(B, T // tb),
        in_specs=[
            pl.BlockSpec((1, 1, tb), lambda i, j: (i, 0, j)),
            pl.BlockSpec((1, d, tb), lambda i, j: (i, 0, j)),
            pl.BlockSpec((CB, d), lambda i, j: (0, 0)),
        ],
        out_specs=[
            pl.BlockSpec((1, d, tb), lambda i, j: (i, 0, j)),
            pl.BlockSpec((1, 1, tb), lambda i, j: (i, 0, j)),
            pl.BlockSpec((1, 1, tb), lambda i, j: (i, 0, j)),
        ],
        out_shape=[
            jax.ShapeDtypeStruct((B, d, T), jnp.float32),
            jax.ShapeDtypeStruct((B, 1, T), jnp.int32),
            jax.ShapeDtypeStruct((B, 1, T), jnp.float32),
        ],
        compiler_params=pltpu.CompilerParams(
            dimension_semantics=("parallel", "parallel"),
        ),
    )(zsq, z, W)
    indices = idx3.reshape(B, T)
    commit_loss = jnp.sum(lossp) / (B * d * T)
    return (q, indices, commit_loss)


# speedup vs baseline: 1.6511x; 1.3554x over previous
import jax
import jax.numpy as jnp
from jax.experimental import pallas as pl
from jax.experimental.pallas import tpu as pltpu

CB = 1024


def _vq_body(zsq_ref, z_ref, w_ref, q_ref, idx_ref, loss_ref):
    zb = z_ref[0]
    w = w_ref[...]
    t = zb.shape[1]
    s2 = jax.lax.dot_general((-2.0 * w).astype(jnp.bfloat16),
                             zb.astype(jnp.bfloat16),
                             (((1,), (0,)), ((), ())),
                             preferred_element_type=jnp.float32)
    zsq = zsq_ref[0]
    wsq = jnp.sum(w * w, axis=1, keepdims=True)
    d2 = (zsq + wsq) + s2
    dist = jnp.sqrt(jnp.maximum(d2, 0.0))
    m = jnp.min(dist, axis=0, keepdims=True)
    iota_f = jax.lax.broadcasted_iota(jnp.int32, (CB, t), 0).astype(jnp.float32)
    idxf = jnp.min(jnp.where(dist == m, iota_f, float(CB)), axis=0)
    idx = idxf.astype(jnp.int32)
    idx_ref[0, 0, :] = idx
    onehot = (iota_f == idxf[None, :]).astype(jnp.bfloat16)
    w_hi = w.astype(jnp.bfloat16)
    r1 = w - w_hi.astype(jnp.float32)
    w_mid = r1.astype(jnp.bfloat16)
    w_lo = (r1 - w_mid.astype(jnp.float32)).astype(jnp.bfloat16)
    dn = (((0,), (0,)), ((), ()))
    q_hi = jax.lax.dot_general(w_hi, onehot, dn, preferred_element_type=jnp.float32)
    q_mid = jax.lax.dot_general(w_mid, onehot, dn, preferred_element_type=jnp.float32)
    q_lo = jax.lax.dot_general(w_lo, onehot, dn, preferred_element_type=jnp.float32)
    q = (q_hi + q_mid) + q_lo
    q_ref[0] = q
    loss_ref[0, 0, :] = jnp.sum((zb - q) ** 2, axis=0)


@jax.jit
def kernel(z, W):
    B, d, T = z.shape
    z_flat = jnp.transpose(z, (0, 2, 1)).reshape(-1, d)
    zsq = jnp.sum(z_flat * z_flat, axis=1).reshape(B, 1, T)
    tb = 512
    q, idx3, lossp = pl.pallas_call(
        _vq_body,
        grid=(B, T // tb),
        in_specs=[
            pl.BlockSpec((1, 1, tb), lambda i, j: (i, 0, j)),
            pl.BlockSpec((1, d, tb), lambda i, j: (i, 0, j)),
            pl.BlockSpec((CB, d), lambda i, j: (0, 0)),
        ],
        out_specs=[
            pl.BlockSpec((1, d, tb), lambda i, j: (i, 0, j)),
            pl.BlockSpec((1, 1, tb), lambda i, j: (i, 0, j)),
            pl.BlockSpec((1, 1, tb), lambda i, j: (i, 0, j)),
        ],
        out_shape=[
            jax.ShapeDtypeStruct((B, d, T), jnp.float32),
            jax.ShapeDtypeStruct((B, 1, T), jnp.int32),
            jax.ShapeDtypeStruct((B, 1, T), jnp.float32),
        ],
        compiler_params=pltpu.CompilerParams(
            dimension_semantics=("parallel", "parallel"),
        ),
    )(zsq, z, W)
    indices = idx3.reshape(B, T)
    commit_loss = jnp.sum(lossp) / (B * d * T)
    return (q, indices, commit_loss)


# speedup vs baseline: 1.9217x; 1.1639x over previous
import jax
import jax.numpy as jnp
from jax.experimental import pallas as pl
from jax.experimental.pallas import tpu as pltpu

CB = 1024


def _vq_body(zsq_ref, z_ref, w_ref, q_ref, idx_ref, loss_ref):
    zb = z_ref[0]
    w = w_ref[...]
    t = zb.shape[1]
    s2 = jax.lax.dot_general((-2.0 * w).astype(jnp.bfloat16),
                             zb.astype(jnp.bfloat16),
                             (((1,), (0,)), ((), ())),
                             preferred_element_type=jnp.float32)
    zsq = zsq_ref[0]
    wsq = jnp.sum(w * w, axis=1, keepdims=True)
    d2 = (zsq + wsq) + s2
    dist = jnp.sqrt(jnp.maximum(d2, 0.0))
    m = jnp.min(dist, axis=0, keepdims=True)
    iota_f = jax.lax.broadcasted_iota(jnp.int32, (CB, t), 0).astype(jnp.float32)
    idxf = jnp.min(jnp.where(dist == m, iota_f, float(CB)), axis=0)
    idx = idxf.astype(jnp.int32)
    idx_ref[0, 0, :] = idx
    onehot = (iota_f == idxf[None, :]).astype(jnp.bfloat16)
    w_hi = w.astype(jnp.bfloat16)
    r1 = w - w_hi.astype(jnp.float32)
    w_mid = r1.astype(jnp.bfloat16)
    w_lo = (r1 - w_mid.astype(jnp.float32)).astype(jnp.bfloat16)
    dn = (((0,), (0,)), ((), ()))
    q_hi = jax.lax.dot_general(w_hi, onehot, dn, preferred_element_type=jnp.float32)
    q_mid = jax.lax.dot_general(w_mid, onehot, dn, preferred_element_type=jnp.float32)
    q_lo = jax.lax.dot_general(w_lo, onehot, dn, preferred_element_type=jnp.float32)
    q = (q_hi + q_mid) + q_lo
    q_ref[0] = q
    loss_ref[0, 0, :] = jnp.sum((zb - q) ** 2, axis=0)


@jax.jit
def kernel(z, W):
    B, d, T = z.shape
    z_flat = jnp.transpose(z, (0, 2, 1)).reshape(-1, d)
    zsq = jnp.sum(z_flat * z_flat, axis=1).reshape(B, 1, T)
    tb = 2048
    q, idx3, lossp = pl.pallas_call(
        _vq_body,
        grid=(B, T // tb),
        in_specs=[
            pl.BlockSpec((1, 1, tb), lambda i, j: (i, 0, j)),
            pl.BlockSpec((1, d, tb), lambda i, j: (i, 0, j)),
            pl.BlockSpec((CB, d), lambda i, j: (0, 0)),
        ],
        out_specs=[
            pl.BlockSpec((1, d, tb), lambda i, j: (i, 0, j)),
            pl.BlockSpec((1, 1, tb), lambda i, j: (i, 0, j)),
            pl.BlockSpec((1, 1, tb), lambda i, j: (i, 0, j)),
        ],
        out_shape=[
            jax.ShapeDtypeStruct((B, d, T), jnp.float32),
            jax.ShapeDtypeStruct((B, 1, T), jnp.int32),
            jax.ShapeDtypeStruct((B, 1, T), jnp.float32),
        ],
        compiler_params=pltpu.CompilerParams(
            dimension_semantics=("parallel", "parallel"),
        ),
    )(zsq, z, W)
    indices = idx3.reshape(B, T)
    commit_loss = jnp.sum(lossp) / (B * d * T)
    return (q, indices, commit_loss)
